# Initial kernel scaffold; baseline (speedup 1.0000x reference)
import jax
import jax.numpy as jnp
from jax.experimental import pallas as pl


def kernel(x, edge_index, W1, a_src1, a_dst1, b1, W2, a_src2, a_dst2, b2):
    raise NotImplementedError("write your pallas kernel here")


# speedup vs baseline: 16.3258x; 16.3258x over previous
import functools

import jax
import jax.numpy as jnp
from jax import lax
from jax.experimental import pallas as pl
from jax.experimental.pallas import tpu as pltpu
from jax.experimental.pallas import tpu_sc as plsc

N = 10000
E = 320000
D = 128
H = 8
HD = H * D

NC = 2
NS = 16
NW = NC * NS
EPW = E // NW
CA = 80
CB = 16
RPT = 640
RPT_LAST = N - (NS - 1) * RPT

f32 = jnp.float32


def _for_tile_slice(s, fn):
    off = pl.multiple_of(s * RPT, 8)
    off_last = (NS - 1) * RPT

    @pl.when(s < NS - 1)
    def _():
        fn(off, RPT)

    @pl.when(s == NS - 1)
    def _():
        fn(off_last, RPT_LAST)


def _tile_slice_copy(s, src_fn, dst_fn):
    _for_tile_slice(
        s, lambda o, n: pltpu.sync_copy(src_fn(o, n), dst_fn(o, n)))


def _dense_body(x, w_ref, as_ref, ad_ref, h_ref, asp_ref, adp_ref, rows):
    h = jnp.dot(x, w_ref[...], preferred_element_type=f32)
    h_ref[...] = h
    h3 = h.reshape(rows, H, D)
    ssrc = jnp.sum(h3 * as_ref[...][None], axis=-1)
    sdst = jnp.sum(h3 * ad_ref[...][None], axis=-1)
    z = jnp.zeros_like(ssrc)
    asp_ref[...] = jnp.concatenate([ssrc, z], axis=-1)
    adp_ref[...] = jnp.concatenate([sdst, z], axis=-1)


def _dense_entry(x, W, a_src, a_dst):
    R = 1000

    def body(x_ref, w_ref, as_ref, ad_ref, h_ref, asp_ref, adp_ref):
        _dense_body(x_ref[...], w_ref, as_ref, ad_ref, h_ref, asp_ref,
                    adp_ref, R)

    return pl.pallas_call(
        body,
        grid=(N // R,),
        in_specs=[
            pl.BlockSpec((R, D), lambda i: (i, 0)),
            pl.BlockSpec((D, HD), lambda i: (0, 0)),
            pl.BlockSpec((H, D), lambda i: (0, 0)),
            pl.BlockSpec((H, D), lambda i: (0, 0)),
        ],
        out_specs=[
            pl.BlockSpec((R, HD), lambda i: (i, 0)),
            pl.BlockSpec((R, 16), lambda i: (i, 0)),
            pl.BlockSpec((R, 16), lambda i: (i, 0)),
        ],
        out_shape=[
            jax.ShapeDtypeStruct((N, HD), f32),
            jax.ShapeDtypeStruct((N, 16), f32),
            jax.ShapeDtypeStruct((N, 16), f32),
        ],
    )(x, W, a_src, a_dst)


def _dense_combine(p0, p1, b2d, W, a_src, a_dst):
    R = 1000

    def body(p0_ref, p1_ref, b_ref, w_ref, as_ref, ad_ref,
             h_ref, asp_ref, adp_ref):
        v = p0_ref[...] + p1_ref[...] + b_ref[...]
        x = jnp.where(v > 0, v, jnp.exp(v) - 1.0)
        _dense_body(x, w_ref, as_ref, ad_ref, h_ref, asp_ref, adp_ref, R)

    return pl.pallas_call(
        body,
        grid=(N // R,),
        in_specs=[
            pl.BlockSpec((R, D), lambda i: (i, 0)),
            pl.BlockSpec((R, D), lambda i: (i, 0)),
            pl.BlockSpec((1, D), lambda i: (0, 0)),
            pl.BlockSpec((D, HD), lambda i: (0, 0)),
            pl.BlockSpec((H, D), lambda i: (0, 0)),
            pl.BlockSpec((H, D), lambda i: (0, 0)),
        ],
        out_specs=[
            pl.BlockSpec((R, HD), lambda i: (i, 0)),
            pl.BlockSpec((R, 16), lambda i: (i, 0)),
            pl.BlockSpec((R, 16), lambda i: (i, 0)),
        ],
        out_shape=[
            jax.ShapeDtypeStruct((N, HD), f32),
            jax.ShapeDtypeStruct((N, 16), f32),
            jax.ShapeDtypeStruct((N, 16), f32),
        ],
    )(p0, p1, b2d, W, a_src, a_dst)


def _final_combine(p0, p1, b2d):
    R = 1000

    def body(p0_ref, p1_ref, b_ref, y_ref):
        v = p0_ref[...] + p1_ref[...] + b_ref[...]
        y_ref[...] = jnp.where(v > 0, v, jnp.exp(v) - 1.0)

    return pl.pallas_call(
        body,
        grid=(N // R,),
        in_specs=[
            pl.BlockSpec((R, D), lambda i: (i, 0)),
            pl.BlockSpec((R, D), lambda i: (i, 0)),
            pl.BlockSpec((1, D), lambda i: (0, 0)),
        ],
        out_specs=pl.BlockSpec((R, D), lambda i: (i, 0)),
        out_shape=jax.ShapeDtypeStruct((N, D), f32),
    )(p0, p1, b2d)


def _sc_pass_a(src_e, dst_e, asp, adp, m16, z16):
    mesh = plsc.VectorSubcoreMesh(core_axis_name="c", subcore_axis_name="s")

    @functools.partial(
        pl.kernel,
        compiler_params=pltpu.CompilerParams(use_tc_tiling_on_sc=False),
        out_type=(
            jax.ShapeDtypeStruct((E, 16), f32),
            jax.ShapeDtypeStruct((NC, N, 16), f32),
        ),
        mesh=mesh,
        scratch_types=[
            pltpu.VMEM((CA,), jnp.int32),
            pltpu.VMEM((CA,), jnp.int32),
            pltpu.VMEM((CA, 16), f32),
            pltpu.VMEM((CA, 16), f32),
            pltpu.VMEM((CA, 16), f32),
            pltpu.VMEM((16,), f32),
            pltpu.VMEM_SHARED((N, 16), f32),
            pltpu.VMEM_SHARED((N, 16), f32),
            pltpu.VMEM_SHARED((N, 16), f32),
            pltpu.SemaphoreType.DMA,
            pltpu.SemaphoreType.DMA,
        ],
    )
    def kern(src_hbm, dst_hbm, asp_hbm, adp_hbm, m_hbm, z_hbm,
             ex_hbm, den_hbm,
             src_v, dst_v, as_v, ad_v, ex_v, m_v,
             asp_sp, adp_sp, den_sp, sem1, sem2):
        c = lax.axis_index("c")
        s = lax.axis_index("s")
        wid = s * NC + c
        base = wid * EPW
        _tile_slice_copy(s, lambda o, n: asp_hbm.at[pl.ds(o, n)],
                         lambda o, n: asp_sp.at[pl.ds(o, n)])
        _tile_slice_copy(s, lambda o, n: adp_hbm.at[pl.ds(o, n)],
                         lambda o, n: adp_sp.at[pl.ds(o, n)])
        _tile_slice_copy(s, lambda o, n: z_hbm.at[pl.ds(o, n)],
                         lambda o, n: den_sp.at[pl.ds(o, n)])
        pltpu.sync_copy(m_hbm, m_v)
        plsc.subcore_barrier()
        mvec = m_v[...]

        def chunk(i, carry):
            off = base + i * CA
            pltpu.sync_copy(src_hbm.at[pl.ds(off, CA)], src_v)
            pltpu.sync_copy(dst_hbm.at[pl.ds(off, CA)], dst_v)
            cp1 = pltpu.async_copy(asp_sp.at[src_v], as_v, sem1)
            cp2 = pltpu.async_copy(adp_sp.at[dst_v], ad_v, sem2)
            cp1.wait()
            cp2.wait()

            def ebody(j, carry2):
                e = as_v[j, :] + ad_v[j, :]
                e = jnp.maximum(e, 0.2 * e) - mvec
                ex_v[j, :] = jnp.exp(e)
                return carry2

            lax.fori_loop(0, CA, ebody, 0)
            pltpu.sync_copy(ex_v, ex_hbm.at[pl.ds(off, CA)])
            pltpu.sync_copy(ex_v, den_sp.at[dst_v], add=True)
            return carry

        lax.fori_loop(0, EPW // CA, chunk, 0)
        plsc.subcore_barrier()
        _tile_slice_copy(s, lambda o, n: den_sp.at[pl.ds(o, n)],
                         lambda o, n: den_hbm.at[c].at[pl.ds(o, n)])

    return kern(src_e, dst_e, asp, adp, m16, z16)


def _sc_pass_b(src_e, dst_e, ex, d0, d1, h, z128):
    mesh = plsc.VectorSubcoreMesh(core_axis_name="c", subcore_axis_name="s")

    @functools.partial(
        pl.kernel,
        compiler_params=pltpu.CompilerParams(use_tc_tiling_on_sc=False),
        out_type=jax.ShapeDtypeStruct((NC, N, D), f32),
        mesh=mesh,
        scratch_types=[
            pltpu.VMEM((CB,), jnp.int32),
            pltpu.VMEM((CB,), jnp.int32),
            pltpu.VMEM((CB, 16), f32),
            pltpu.VMEM((CB, 16), f32),
            pltpu.VMEM((CB, HD), f32),
            pltpu.VMEM((CB, D), f32),
            pltpu.VMEM((80, 16), f32),
            pltpu.VMEM((80, 16), f32),
            pltpu.VMEM_SHARED((N, 16), f32),
            pltpu.VMEM_SHARED((N, D), f32),
            pltpu.SemaphoreType.DMA,
            pltpu.SemaphoreType.DMA,
        ],
    )
    def kern(src_hbm, dst_hbm, ex_hbm, d0_hbm, d1_hbm, h_hbm, z_hbm,
             out_hbm,
             src_v, dst_v, ex_v, dn_v, h_v, ct_v, t0_v, t1_v,
             den_sp, acc_sp, sem1, sem2):
        c = lax.axis_index("c")
        s = lax.axis_index("s")
        wid = s * NC + c
        base = wid * EPW
        def den_combine(o, n):
            def sub(k, carry):
                oo = pl.multiple_of(o + k * 80, 8)
                pltpu.sync_copy(d0_hbm.at[pl.ds(oo, 80)], t0_v)
                pltpu.sync_copy(d1_hbm.at[pl.ds(oo, 80)], t1_v)

                def add_row(r, carry2):
                    t0_v[r, :] = t0_v[r, :] + t1_v[r, :]
                    return carry2

                lax.fori_loop(0, 80, add_row, 0)
                pltpu.sync_copy(t0_v, den_sp.at[pl.ds(oo, 80)])
                return carry

            lax.fori_loop(0, n // 80, sub, 0)

        _for_tile_slice(s, den_combine)
        _tile_slice_copy(s, lambda o, n: z_hbm.at[pl.ds(o, n)],
                         lambda o, n: acc_sp.at[pl.ds(o, n)])
        plsc.subcore_barrier()

        def chunk(i, carry):
            off = base + i * CB
            pltpu.sync_copy(src_hbm.at[pl.ds(off, CB)], src_v)
            pltpu.sync_copy(dst_hbm.at[pl.ds(off, CB)], dst_v)
            pltpu.sync_copy(ex_hbm.at[pl.ds(off, CB)], ex_v)
            cp1 = pltpu.async_copy(den_sp.at[dst_v], dn_v, sem1)
            cp3 = pltpu.async_copy(h_hbm.at[src_v], h_v, sem2)
            cp1.wait()
            cp3.wait()

            def ebody(j, carry2):
                al = ex_v[j, :] / (dn_v[j, :] + 1e-16)
                al = al * (1.0 / H)
                acc = [None] * 8
                for head in range(H):
                    sv = al[head]
                    for blk in range(8):
                        seg = h_v[j, pl.ds(head * D + blk * 16, 16)]
                        if head == 0:
                            acc[blk] = sv * seg
                        else:
                            acc[blk] = acc[blk] + sv * seg
                for blk in range(8):
                    ct_v[j, pl.ds(blk * 16, 16)] = acc[blk]
                return carry2

            lax.fori_loop(0, CB, ebody, 0)
            pltpu.sync_copy(ct_v, acc_sp.at[dst_v], add=True)
            return carry

        lax.fori_loop(0, EPW // CB, chunk, 0)
        plsc.subcore_barrier()
        _tile_slice_copy(s, lambda o, n: acc_sp.at[pl.ds(o, n)],
                         lambda o, n: out_hbm.at[c].at[pl.ds(o, n)])

    return kern(src_e, dst_e, ex, d0, d1, h, z128)


def _layer_sc(h, asp, adp, src_e, dst_e):
    m8 = jnp.maximum(jnp.max(asp[:, :8], axis=0) + jnp.max(adp[:, :8], axis=0),
                     0.0)
    m16 = jnp.concatenate([m8, jnp.zeros((8,), f32)])
    z16 = jnp.zeros((N, 16), f32)
    ex, den = _sc_pass_a(src_e, dst_e, asp, adp, m16, z16)
    z128 = jnp.zeros((N, D), f32)
    outp = _sc_pass_b(src_e, dst_e, ex, den[0], den[1], h, z128)
    return outp


def kernel(x, edge_index, W1, a_src1, a_dst1, b1, W2, a_src2, a_dst2, b2):
    src_e = edge_index[0]
    dst_e = edge_index[1]
    h1, asp1, adp1 = _dense_entry(x, W1, a_src1, a_dst1)
    p1 = _layer_sc(h1, asp1, adp1, src_e, dst_e)
    h2, asp2, adp2 = _dense_combine(p1[0], p1[1], b1.reshape(1, D),
                                    W2, a_src2, a_dst2)
    p2 = _layer_sc(h2, asp2, adp2, src_e, dst_e)
    return _final_combine(p2[0], p2[1], b2.reshape(1, D))


# speedup vs baseline: 33.1586x; 2.0311x over previous
import functools

import jax
import jax.numpy as jnp
from jax import lax
from jax.experimental import pallas as pl
from jax.experimental.pallas import tpu as pltpu
from jax.experimental.pallas import tpu_sc as plsc

N = 10000
E = 320000
D = 128
H = 8
HD = H * D

NC = 2
NS = 16
NW = NC * NS
EPW = E // NW
CA = 80
CB = 16
RPT = 640
RPT_LAST = N - (NS - 1) * RPT

f32 = jnp.float32


def _for_tile_slice(s, fn):
    off = pl.multiple_of(s * RPT, 8)
    off_last = (NS - 1) * RPT

    @pl.when(s < NS - 1)
    def _():
        fn(off, RPT)

    @pl.when(s == NS - 1)
    def _():
        fn(off_last, RPT_LAST)


def _tile_slice_copy(s, src_fn, dst_fn):
    _for_tile_slice(
        s, lambda o, n: pltpu.sync_copy(src_fn(o, n), dst_fn(o, n)))


def _dense_body(x, w_ref, as_ref, ad_ref, h_ref, asp_ref, adp_ref, rows):
    h = jnp.dot(x, w_ref[...], preferred_element_type=f32)
    h_ref[...] = h
    h3 = h.reshape(rows, H, D)
    ssrc = jnp.sum(h3 * as_ref[...][None], axis=-1)
    sdst = jnp.sum(h3 * ad_ref[...][None], axis=-1)
    z = jnp.zeros_like(ssrc)
    asp_ref[...] = jnp.concatenate([ssrc, z], axis=-1)
    adp_ref[...] = jnp.concatenate([sdst, z], axis=-1)


def _dense_entry(x, W, a_src, a_dst):
    R = 1000

    def body(x_ref, w_ref, as_ref, ad_ref, h_ref, asp_ref, adp_ref):
        _dense_body(x_ref[...], w_ref, as_ref, ad_ref, h_ref, asp_ref,
                    adp_ref, R)

    return pl.pallas_call(
        body,
        grid=(N // R,),
        in_specs=[
            pl.BlockSpec((R, D), lambda i: (i, 0)),
            pl.BlockSpec((D, HD), lambda i: (0, 0)),
            pl.BlockSpec((H, D), lambda i: (0, 0)),
            pl.BlockSpec((H, D), lambda i: (0, 0)),
        ],
        out_specs=[
            pl.BlockSpec((R, HD), lambda i: (i, 0)),
            pl.BlockSpec((R, 16), lambda i: (i, 0)),
            pl.BlockSpec((R, 16), lambda i: (i, 0)),
        ],
        out_shape=[
            jax.ShapeDtypeStruct((N, HD), f32),
            jax.ShapeDtypeStruct((N, 16), f32),
            jax.ShapeDtypeStruct((N, 16), f32),
        ],
    )(x, W, a_src, a_dst)


def _dense_combine(p0, p1, b2d, W, a_src, a_dst):
    R = 1000

    def body(p0_ref, p1_ref, b_ref, w_ref, as_ref, ad_ref,
             h_ref, asp_ref, adp_ref):
        v = p0_ref[...] + p1_ref[...] + b_ref[...]
        x = jnp.where(v > 0, v, jnp.exp(v) - 1.0)
        _dense_body(x, w_ref, as_ref, ad_ref, h_ref, asp_ref, adp_ref, R)

    return pl.pallas_call(
        body,
        grid=(N // R,),
        in_specs=[
            pl.BlockSpec((R, D), lambda i: (i, 0)),
            pl.BlockSpec((R, D), lambda i: (i, 0)),
            pl.BlockSpec((1, D), lambda i: (0, 0)),
            pl.BlockSpec((D, HD), lambda i: (0, 0)),
            pl.BlockSpec((H, D), lambda i: (0, 0)),
            pl.BlockSpec((H, D), lambda i: (0, 0)),
        ],
        out_specs=[
            pl.BlockSpec((R, HD), lambda i: (i, 0)),
            pl.BlockSpec((R, 16), lambda i: (i, 0)),
            pl.BlockSpec((R, 16), lambda i: (i, 0)),
        ],
        out_shape=[
            jax.ShapeDtypeStruct((N, HD), f32),
            jax.ShapeDtypeStruct((N, 16), f32),
            jax.ShapeDtypeStruct((N, 16), f32),
        ],
    )(p0, p1, b2d, W, a_src, a_dst)


def _final_combine(p0, p1, b2d):
    R = 1000

    def body(p0_ref, p1_ref, b_ref, y_ref):
        v = p0_ref[...] + p1_ref[...] + b_ref[...]
        y_ref[...] = jnp.where(v > 0, v, jnp.exp(v) - 1.0)

    return pl.pallas_call(
        body,
        grid=(N // R,),
        in_specs=[
            pl.BlockSpec((R, D), lambda i: (i, 0)),
            pl.BlockSpec((R, D), lambda i: (i, 0)),
            pl.BlockSpec((1, D), lambda i: (0, 0)),
        ],
        out_specs=pl.BlockSpec((R, D), lambda i: (i, 0)),
        out_shape=jax.ShapeDtypeStruct((N, D), f32),
    )(p0, p1, b2d)


def _sc_pass_a(src_e, dst_e, asp, adp, m16, z16):
    mesh = plsc.VectorSubcoreMesh(core_axis_name="c", subcore_axis_name="s")

    @functools.partial(
        pl.kernel,
        compiler_params=pltpu.CompilerParams(use_tc_tiling_on_sc=False),
        out_type=(
            jax.ShapeDtypeStruct((E, 16), f32),
            jax.ShapeDtypeStruct((NC, N, 16), f32),
        ),
        mesh=mesh,
        scratch_types=[
            pltpu.VMEM((CA,), jnp.int32),
            pltpu.VMEM((CA,), jnp.int32),
            pltpu.VMEM((CA, 16), f32),
            pltpu.VMEM((CA, 16), f32),
            pltpu.VMEM((CA, 16), f32),
            pltpu.VMEM((16,), f32),
            pltpu.VMEM_SHARED((N, 16), f32),
            pltpu.VMEM_SHARED((N, 16), f32),
            pltpu.VMEM_SHARED((N, 16), f32),
            pltpu.SemaphoreType.DMA,
            pltpu.SemaphoreType.DMA,
        ],
    )
    def kern(src_hbm, dst_hbm, asp_hbm, adp_hbm, m_hbm, z_hbm,
             ex_hbm, den_hbm,
             src_v, dst_v, as_v, ad_v, ex_v, m_v,
             asp_sp, adp_sp, den_sp, sem1, sem2):
        c = lax.axis_index("c")
        s = lax.axis_index("s")
        wid = s * NC + c
        base = wid * EPW
        _tile_slice_copy(s, lambda o, n: asp_hbm.at[pl.ds(o, n)],
                         lambda o, n: asp_sp.at[pl.ds(o, n)])
        _tile_slice_copy(s, lambda o, n: adp_hbm.at[pl.ds(o, n)],
                         lambda o, n: adp_sp.at[pl.ds(o, n)])
        _tile_slice_copy(s, lambda o, n: z_hbm.at[pl.ds(o, n)],
                         lambda o, n: den_sp.at[pl.ds(o, n)])
        pltpu.sync_copy(m_hbm, m_v)
        plsc.subcore_barrier()
        mvec = m_v[...]

        def chunk(i, carry):
            off = base + i * CA
            pltpu.sync_copy(src_hbm.at[pl.ds(off, CA)], src_v)
            pltpu.sync_copy(dst_hbm.at[pl.ds(off, CA)], dst_v)
            cp1 = pltpu.async_copy(asp_sp.at[src_v], as_v, sem1)
            cp2 = pltpu.async_copy(adp_sp.at[dst_v], ad_v, sem2)
            cp1.wait()
            cp2.wait()

            def ebody(j, carry2):
                e = as_v[j, :] + ad_v[j, :]
                e = jnp.maximum(e, 0.2 * e) - mvec
                ex_v[j, :] = jnp.exp(e)
                return carry2

            lax.fori_loop(0, CA, ebody, 0)
            pltpu.sync_copy(ex_v, ex_hbm.at[pl.ds(off, CA)])
            pltpu.sync_copy(ex_v, den_sp.at[dst_v], add=True)
            return carry

        lax.fori_loop(0, EPW // CA, chunk, 0)
        plsc.subcore_barrier()
        _tile_slice_copy(s, lambda o, n: den_sp.at[pl.ds(o, n)],
                         lambda o, n: den_hbm.at[c].at[pl.ds(o, n)])

    return kern(src_e, dst_e, asp, adp, m16, z16)


def _sc_pass_b(src_e, dst_e, ex, d0, d1, h, z128):
    mesh = plsc.VectorSubcoreMesh(core_axis_name="c", subcore_axis_name="s")

    @functools.partial(
        pl.kernel,
        compiler_params=pltpu.CompilerParams(use_tc_tiling_on_sc=False),
        out_type=jax.ShapeDtypeStruct((NC, N, D), f32),
        mesh=mesh,
        scratch_types=[
            [pltpu.VMEM((CB,), jnp.int32)] * 2,
            [pltpu.VMEM((CB,), jnp.int32)] * 2,
            [pltpu.VMEM((CB, 16), f32)] * 2,
            [pltpu.VMEM((CB, 16), f32)] * 2,
            [pltpu.VMEM((CB, HD), f32)] * 2,
            pltpu.VMEM((CB, D), f32),
            pltpu.VMEM((CB,), jnp.int32),
            pltpu.VMEM((16, 16), f32),
            pltpu.VMEM((16, 16), f32),
            pltpu.VMEM_SHARED((N, 16), f32),
            pltpu.VMEM_SHARED((N, D), f32),
            [pltpu.SemaphoreType.DMA] * 2,
            [pltpu.SemaphoreType.DMA] * 2,
            [pltpu.SemaphoreType.DMA] * 2,
            [pltpu.SemaphoreType.DMA] * 2,
            [pltpu.SemaphoreType.DMA] * 2,
        ],
    )
    def kern(src_hbm, dst_hbm, ex_hbm, d0_hbm, d1_hbm, h_hbm, z_hbm,
             out_hbm,
             srcs, dsts, exs, dns, hs, ct_v, dsc_v, t0_v, t1_v,
             den_sp, acc_sp, sem_src, sem_dst, sem_ex, sem_dn, sem_h):
        c = lax.axis_index("c")
        s = lax.axis_index("s")
        wid = s * NC + c
        base = wid * EPW
        nch = EPW // CB
        last = nch - 1

        def den_combine(o, n):
            def sub(k, carry):
                oo = pl.multiple_of(o + k * 16, 8)
                pltpu.sync_copy(d0_hbm.at[pl.ds(oo, 16)], t0_v)
                pltpu.sync_copy(d1_hbm.at[pl.ds(oo, 16)], t1_v)

                def add_row(r, carry2):
                    t0_v[r, :] = t0_v[r, :] + t1_v[r, :]
                    return carry2

                lax.fori_loop(0, 16, add_row, 0)
                pltpu.sync_copy(t0_v, den_sp.at[pl.ds(oo, 16)])
                return carry

            lax.fori_loop(0, n // 16, sub, 0)

        _for_tile_slice(s, den_combine)
        _tile_slice_copy(s, lambda o, n: z_hbm.at[pl.ds(o, n)],
                         lambda o, n: acc_sp.at[pl.ds(o, n)])
        plsc.subcore_barrier()

        def coff(g):
            return pl.multiple_of(base + jnp.minimum(g, last) * CB, 8)

        def sd_issue(g, k):
            off = coff(g)
            pltpu.async_copy(src_hbm.at[pl.ds(off, CB)], srcs[k], sem_src[k])
            pltpu.async_copy(dst_hbm.at[pl.ds(off, CB)], dsts[k], sem_dst[k])

        def sd_wait(k):
            pltpu.make_async_copy(src_hbm.at[pl.ds(0, CB)], srcs[k],
                                  sem_src[k]).wait()
            pltpu.make_async_copy(dst_hbm.at[pl.ds(0, CB)], dsts[k],
                                  sem_dst[k]).wait()

        def ex_issue(g, k):
            pltpu.async_copy(ex_hbm.at[pl.ds(coff(g), CB)], exs[k], sem_ex[k])

        def ex_wait(k):
            pltpu.make_async_copy(ex_hbm.at[pl.ds(0, CB)], exs[k],
                                  sem_ex[k]).wait()

        def g_issue(k):
            pltpu.async_copy(den_sp.at[dsts[k]], dns[k], sem_dn[k])
            pltpu.async_copy(h_hbm.at[srcs[k]], hs[k], sem_h[k])

        def g_wait(k):
            pltpu.make_async_copy(den_sp.at[dsts[k]], dns[k], sem_dn[k]).wait()
            pltpu.make_async_copy(h_hbm.at[srcs[k]], hs[k], sem_h[k]).wait()

        def compute(k):
            ex_v, dn_v, h_v = exs[k], dns[k], hs[k]

            def ebody(j, carry2):
                al = ex_v[j, :] / (dn_v[j, :] + 1e-16)
                al = al * (1.0 / H)
                acc = [None] * 8
                for head in range(H):
                    sv = al[head]
                    for blk in range(8):
                        seg = h_v[j, pl.ds(head * D + blk * 16, 16)]
                        if head == 0:
                            acc[blk] = sv * seg
                        else:
                            acc[blk] = acc[blk] + sv * seg
                for blk in range(8):
                    ct_v[j, pl.ds(blk * 16, 16)] = acc[blk]
                return carry2

            lax.fori_loop(0, CB, ebody, 0)
            pltpu.sync_copy(ct_v, acc_sp.at[dsc_v], add=True)

        sd_issue(0, 0)
        sd_issue(1, 1)
        ex_issue(0, 0)
        ex_issue(1, 1)
        sd_wait(0)
        g_issue(0)

        def pipe(i, carry):
            g0 = 2 * i
            for k in (0, 1):
                g = g0 + k
                nk = 1 - k
                g_wait(k)
                dsc_v[...] = dsts[k][...]
                sd_wait(nk)
                g_issue(nk)
                sd_issue(g + 2, k)
                ex_wait(k)
                compute(k)
                ex_issue(g + 2, k)
            return carry

        lax.fori_loop(0, (nch - 1) // 2, pipe, 0)
        sd_wait(1)
        ex_wait(1)
        g_wait(0)
        dsc_v[...] = dsts[0][...]
        ex_wait(0)
        compute(0)
        plsc.subcore_barrier()
        _tile_slice_copy(s, lambda o, n: acc_sp.at[pl.ds(o, n)],
                         lambda o, n: out_hbm.at[c].at[pl.ds(o, n)])

    return kern(src_e, dst_e, ex, d0, d1, h, z128)


def _layer_sc(h, asp, adp, src_e, dst_e):
    m8 = jnp.maximum(jnp.max(asp[:, :8], axis=0) + jnp.max(adp[:, :8], axis=0),
                     0.0)
    m16 = jnp.concatenate([m8, jnp.zeros((8,), f32)])
    z16 = jnp.zeros((N, 16), f32)
    ex, den = _sc_pass_a(src_e, dst_e, asp, adp, m16, z16)
    z128 = jnp.zeros((N, D), f32)
    outp = _sc_pass_b(src_e, dst_e, ex, den[0], den[1], h, z128)
    return outp


def kernel(x, edge_index, W1, a_src1, a_dst1, b1, W2, a_src2, a_dst2, b2):
    src_e = edge_index[0]
    dst_e = edge_index[1]
    h1, asp1, adp1 = _dense_entry(x, W1, a_src1, a_dst1)
    p1 = _layer_sc(h1, asp1, adp1, src_e, dst_e)
    h2, asp2, adp2 = _dense_combine(p1[0], p1[1], b1.reshape(1, D),
                                    W2, a_src2, a_dst2)
    p2 = _layer_sc(h2, asp2, adp2, src_e, dst_e)
    return _final_combine(p2[0], p2[1], b2.reshape(1, D))


# speedup vs baseline: 68.2336x; 2.0578x over previous
import functools

import jax
import jax.numpy as jnp
from jax import lax
from jax.experimental import pallas as pl
from jax.experimental.pallas import tpu as pltpu
from jax.experimental.pallas import tpu_sc as plsc

N = 10000
E = 320000
D = 128
H = 8
HD = H * D

NC = 2
NS = 16
NW = NC * NS
EPW = E // NW
CA = 80
CB = 16
RPT = 640
RPT_LAST = N - (NS - 1) * RPT

f32 = jnp.float32


def _for_tile_slice(s, fn):
    off = pl.multiple_of(s * RPT, 8)
    off_last = (NS - 1) * RPT

    @pl.when(s < NS - 1)
    def _():
        fn(off, RPT)

    @pl.when(s == NS - 1)
    def _():
        fn(off_last, RPT_LAST)


def _tile_slice_copy(s, src_fn, dst_fn):
    _for_tile_slice(
        s, lambda o, n: pltpu.sync_copy(src_fn(o, n), dst_fn(o, n)))


def _dense_body(x, w_ref, as_ref, ad_ref, h_ref, asp_ref, adp_ref, rows):
    h = jnp.dot(x, w_ref[...], preferred_element_type=f32)
    h_ref[...] = h.astype(jnp.bfloat16)
    h3 = h.reshape(rows, H, D)
    ssrc = jnp.sum(h3 * as_ref[...][None], axis=-1)
    sdst = jnp.sum(h3 * ad_ref[...][None], axis=-1)
    z = jnp.zeros_like(ssrc)
    asp_ref[...] = jnp.concatenate([ssrc, z], axis=-1)
    adp_ref[...] = jnp.concatenate([sdst, z], axis=-1)


def _dense_entry(x, W, a_src, a_dst):
    R = 2000

    def body(x_ref, w_ref, as_ref, ad_ref, h_ref, asp_ref, adp_ref):
        _dense_body(x_ref[...], w_ref, as_ref, ad_ref, h_ref, asp_ref,
                    adp_ref, R)

    return pl.pallas_call(
        body,
        grid=(N // R,),
        in_specs=[
            pl.BlockSpec((R, D), lambda i: (i, 0)),
            pl.BlockSpec((D, HD), lambda i: (0, 0)),
            pl.BlockSpec((H, D), lambda i: (0, 0)),
            pl.BlockSpec((H, D), lambda i: (0, 0)),
        ],
        out_specs=[
            pl.BlockSpec((R, HD), lambda i: (i, 0)),
            pl.BlockSpec((R, 16), lambda i: (i, 0)),
            pl.BlockSpec((R, 16), lambda i: (i, 0)),
        ],
        out_shape=[
            jax.ShapeDtypeStruct((N, HD), jnp.bfloat16),
            jax.ShapeDtypeStruct((N, 16), f32),
            jax.ShapeDtypeStruct((N, 16), f32),
        ],
    )(x, W, a_src, a_dst)


def _dense_combine(p0, p1, b2d, W, a_src, a_dst):
    R = 2000

    def body(p0_ref, p1_ref, b_ref, w_ref, as_ref, ad_ref,
             h_ref, asp_ref, adp_ref):
        v = p0_ref[...] + p1_ref[...] + b_ref[...]
        x = jnp.where(v > 0, v, jnp.exp(v) - 1.0)
        _dense_body(x, w_ref, as_ref, ad_ref, h_ref, asp_ref, adp_ref, R)

    return pl.pallas_call(
        body,
        grid=(N // R,),
        in_specs=[
            pl.BlockSpec((R, D), lambda i: (i, 0)),
            pl.BlockSpec((R, D), lambda i: (i, 0)),
            pl.BlockSpec((1, D), lambda i: (0, 0)),
            pl.BlockSpec((D, HD), lambda i: (0, 0)),
            pl.BlockSpec((H, D), lambda i: (0, 0)),
            pl.BlockSpec((H, D), lambda i: (0, 0)),
        ],
        out_specs=[
            pl.BlockSpec((R, HD), lambda i: (i, 0)),
            pl.BlockSpec((R, 16), lambda i: (i, 0)),
            pl.BlockSpec((R, 16), lambda i: (i, 0)),
        ],
        out_shape=[
            jax.ShapeDtypeStruct((N, HD), jnp.bfloat16),
            jax.ShapeDtypeStruct((N, 16), f32),
            jax.ShapeDtypeStruct((N, 16), f32),
        ],
    )(p0, p1, b2d, W, a_src, a_dst)


def _final_combine(p0, p1, b2d):
    R = 2000

    def body(p0_ref, p1_ref, b_ref, y_ref):
        v = p0_ref[...] + p1_ref[...] + b_ref[...]
        y_ref[...] = jnp.where(v > 0, v, jnp.exp(v) - 1.0)

    return pl.pallas_call(
        body,
        grid=(N // R,),
        in_specs=[
            pl.BlockSpec((R, D), lambda i: (i, 0)),
            pl.BlockSpec((R, D), lambda i: (i, 0)),
            pl.BlockSpec((1, D), lambda i: (0, 0)),
        ],
        out_specs=pl.BlockSpec((R, D), lambda i: (i, 0)),
        out_shape=jax.ShapeDtypeStruct((N, D), f32),
    )(p0, p1, b2d)


def _sc_pass_a(ei, asp, adp, m16, z16):
    mesh = plsc.VectorSubcoreMesh(core_axis_name="c", subcore_axis_name="s")

    @functools.partial(
        pl.kernel,
        compiler_params=pltpu.CompilerParams(use_tc_tiling_on_sc=False,
                                             needs_layout_passes=False),
        out_type=(
            jax.ShapeDtypeStruct((E, 16), f32),
            jax.ShapeDtypeStruct((NC, N, 16), f32),
        ),
        mesh=mesh,
        scratch_types=[
            [pltpu.VMEM((CA,), jnp.int32)] * 2,
            [pltpu.VMEM((CA,), jnp.int32)] * 2,
            [pltpu.VMEM((CA, 16), f32)] * 2,
            [pltpu.VMEM((CA, 16), f32)] * 2,
            [pltpu.VMEM((CA, 16), f32)] * 2,
            [pltpu.VMEM((CA,), jnp.int32)] * 2,
            pltpu.VMEM((16,), f32),
            pltpu.VMEM_SHARED((N, 16), f32),
            pltpu.VMEM_SHARED((N, 16), f32),
            pltpu.VMEM_SHARED((N, 16), f32),
            [pltpu.SemaphoreType.DMA] * 2,
            [pltpu.SemaphoreType.DMA] * 2,
            [pltpu.SemaphoreType.DMA] * 2,
            [pltpu.SemaphoreType.DMA] * 2,
            [pltpu.SemaphoreType.DMA] * 2,
            [pltpu.SemaphoreType.DMA] * 2,
        ],
    )
    def kern(ei_hbm, asp_hbm, adp_hbm, m_hbm, z_hbm,
             ex_hbm, den_hbm,
             srcs, dsts, ass, ads, exs, dscs, m_v,
             asp_sp, adp_sp, den_sp,
             sem_src, sem_dst, sem_as, sem_ad, sem_w, sem_sc):
        src_hbm = ei_hbm.at[0]
        dst_hbm = ei_hbm.at[1]
        c = lax.axis_index("c")
        s = lax.axis_index("s")
        wid = s * NC + c
        base = wid * EPW
        nch = EPW // CA
        last = nch - 1
        _tile_slice_copy(s, lambda o, n: asp_hbm.at[pl.ds(o, n)],
                         lambda o, n: asp_sp.at[pl.ds(o, n)])
        _tile_slice_copy(s, lambda o, n: adp_hbm.at[pl.ds(o, n)],
                         lambda o, n: adp_sp.at[pl.ds(o, n)])
        _tile_slice_copy(s, lambda o, n: z_hbm.at[pl.ds(o, n)],
                         lambda o, n: den_sp.at[pl.ds(o, n)])
        pltpu.sync_copy(m_hbm, m_v)
        plsc.subcore_barrier()
        mvec = m_v[...]

        def coff(g):
            return pl.multiple_of(base + jnp.minimum(g, last) * CA, 8)

        def sd_issue(g, k):
            off = coff(g)
            pltpu.async_copy(src_hbm.at[pl.ds(off, CA)], srcs[k], sem_src[k])
            pltpu.async_copy(dst_hbm.at[pl.ds(off, CA)], dsts[k], sem_dst[k])

        def sd_wait(k):
            pltpu.make_async_copy(src_hbm.at[pl.ds(0, CA)], srcs[k],
                                  sem_src[k]).wait()
            pltpu.make_async_copy(dst_hbm.at[pl.ds(0, CA)], dsts[k],
                                  sem_dst[k]).wait()

        def g_issue(k):
            pltpu.async_copy(asp_sp.at[srcs[k]], ass[k], sem_as[k])
            pltpu.async_copy(adp_sp.at[dsts[k]], ads[k], sem_ad[k])

        def g_wait(k):
            pltpu.make_async_copy(asp_sp.at[srcs[k]], ass[k],
                                  sem_as[k]).wait()
            pltpu.make_async_copy(adp_sp.at[dsts[k]], ads[k],
                                  sem_ad[k]).wait()

        def out_drain(k):
            pltpu.make_async_copy(exs[k], ex_hbm.at[pl.ds(0, CA)],
                                  sem_w[k]).wait()
            pltpu.make_async_copy(exs[k], den_sp.at[dscs[k]],
                                  sem_sc[k]).wait()

        def compute(g, k):
            as_v, ad_v, ex_v = ass[k], ads[k], exs[k]

            @plsc.parallel_loop(0, CA, unroll=2)
            def _(j):
                e = as_v[j, :] + ad_v[j, :]
                e = jnp.maximum(e, 0.2 * e) - mvec
                ex_v[j, :] = jnp.exp(e)

            pltpu.async_copy(ex_v, ex_hbm.at[pl.ds(coff(g), CA)], sem_w[k])
            pltpu.async_copy(ex_v, den_sp.at[dscs[k]], sem_sc[k], add=True)

        sd_issue(0, 0)
        sd_issue(1, 1)
        sd_wait(0)
        g_issue(0)

        def pipe(i, carry):
            g0 = 2 * i
            for k in (0, 1):
                g = g0 + k
                nk = 1 - k
                g_wait(k)
                for q in range(CA // 16):
                    dscs[k][pl.ds(q * 16, 16)] = dsts[k][pl.ds(q * 16, 16)]
                sd_wait(nk)
                g_issue(nk)
                sd_issue(g + 2, k)
                compute(g, k)

                @pl.when(g >= 1)
                def _():
                    out_drain(nk)
            return carry

        lax.fori_loop(0, (nch - 1) // 2, pipe, 0)
        g_wait(0)
        for q in range(CA // 16):
            dscs[0][pl.ds(q * 16, 16)] = dsts[0][pl.ds(q * 16, 16)]
        compute(nch - 1, 0)
        sd_wait(1)
        out_drain(1)
        out_drain(0)
        plsc.subcore_barrier()
        _tile_slice_copy(s, lambda o, n: den_sp.at[pl.ds(o, n)],
                         lambda o, n: den_hbm.at[c].at[pl.ds(o, n)])

    return kern(ei, asp, adp, m16, z16)


def _sc_pass_b(ei, ex, den, h, z128):
    mesh = plsc.VectorSubcoreMesh(core_axis_name="c", subcore_axis_name="s")
    S = 3

    @functools.partial(
        pl.kernel,
        compiler_params=pltpu.CompilerParams(use_tc_tiling_on_sc=False,
                                             needs_layout_passes=False),
        out_type=jax.ShapeDtypeStruct((NC, N, D), f32),
        mesh=mesh,
        scratch_types=[
            [pltpu.VMEM((CB,), jnp.int32)] * S,
            [pltpu.VMEM((CB,), jnp.int32)] * S,
            [pltpu.VMEM((CB, 16), f32)] * S,
            [pltpu.VMEM((CB, 16), f32)] * S,
            [pltpu.VMEM((CB, HD), jnp.bfloat16)] * S,
            [pltpu.VMEM((CB, D), f32)] * S,
            [pltpu.VMEM((CB,), jnp.int32)] * S,
            pltpu.VMEM((16, 16), f32),
            pltpu.VMEM((16, 16), f32),
            pltpu.VMEM_SHARED((N, 16), f32),
            pltpu.VMEM_SHARED((N, D), f32),
            [pltpu.SemaphoreType.DMA] * S,
            [pltpu.SemaphoreType.DMA] * S,
            [pltpu.SemaphoreType.DMA] * S,
            [pltpu.SemaphoreType.DMA] * S,
            [[pltpu.SemaphoreType.DMA] * S] * 2,
            [pltpu.SemaphoreType.DMA] * S,
        ],
    )
    def kern(ei_hbm, ex_hbm, den_hbm, h_hbm, z_hbm,
             out_hbm,
             srcs, dsts, exs, dns, hs, cts, dscs, t0_v, t1_v,
             den_sp, acc_sp, sem_src, sem_dst, sem_ex, sem_dn, sem_h,
             sem_ct):
        src_hbm = ei_hbm.at[0]
        dst_hbm = ei_hbm.at[1]
        c = lax.axis_index("c")
        s = lax.axis_index("s")
        wid = s * NC + c
        base = wid * EPW
        nch = EPW // CB
        last = nch - 1

        def den_combine(o, n):
            def sub(k, carry):
                oo = pl.multiple_of(o + k * 16, 8)
                pltpu.sync_copy(den_hbm.at[0].at[pl.ds(oo, 16)], t0_v)
                pltpu.sync_copy(den_hbm.at[1].at[pl.ds(oo, 16)], t1_v)

                def add_row(r, carry2):
                    t0_v[r, :] = t0_v[r, :] + t1_v[r, :]
                    return carry2

                lax.fori_loop(0, 16, add_row, 0)
                pltpu.sync_copy(t0_v, den_sp.at[pl.ds(oo, 16)])
                return carry

            lax.fori_loop(0, n // 16, sub, 0)

        _for_tile_slice(s, den_combine)
        _tile_slice_copy(s, lambda o, n: z_hbm.at[pl.ds(o, n)],
                         lambda o, n: acc_sp.at[pl.ds(o, n)])
        plsc.subcore_barrier()

        HB = CB // 2

        def coff(g):
            return pl.multiple_of(base + jnp.minimum(g, last) * CB, 8)

        def sd_issue(g, k):
            off = coff(g)
            pltpu.async_copy(src_hbm.at[pl.ds(off, CB)], srcs[k], sem_src[k])
            pltpu.async_copy(dst_hbm.at[pl.ds(off, CB)], dsts[k], sem_dst[k])

        def sd_wait(k):
            pltpu.make_async_copy(src_hbm.at[pl.ds(0, CB)], srcs[k],
                                  sem_src[k]).wait()
            pltpu.make_async_copy(dst_hbm.at[pl.ds(0, CB)], dsts[k],
                                  sem_dst[k]).wait()

        def ex_issue(g, k):
            pltpu.async_copy(ex_hbm.at[pl.ds(coff(g), CB)], exs[k], sem_ex[k])

        def ex_wait(k):
            pltpu.make_async_copy(ex_hbm.at[pl.ds(0, CB)], exs[k],
                                  sem_ex[k]).wait()

        def g_issue(k):
            pltpu.async_copy(den_sp.at[dsts[k]], dns[k], sem_dn[k])
            for q in range(2):
                pltpu.async_copy(h_hbm.at[srcs[k].at[pl.ds(q * HB, HB)]],
                                 hs[k].at[pl.ds(q * HB, HB)], sem_h[q][k])

        def g_wait(k):
            pltpu.make_async_copy(den_sp.at[dsts[k]], dns[k], sem_dn[k]).wait()
            for q in range(2):
                pltpu.make_async_copy(h_hbm.at[srcs[k].at[pl.ds(q * HB, HB)]],
                                      hs[k].at[pl.ds(q * HB, HB)],
                                      sem_h[q][k]).wait()

        def ct_drain(k):
            pltpu.make_async_copy(cts[k], acc_sp.at[dscs[k]],
                                  sem_ct[k]).wait()

        def compute(k):
            ex_v, dn_v, h_v, ct_v = exs[k], dns[k], hs[k], cts[k]

            @plsc.parallel_loop(0, CB, unroll=2)
            def _(j):
                al = ex_v[j, :] / (dn_v[j, :] + 1e-16)
                al = al * (1.0 / H)
                acc = [None] * 4
                for head in range(H):
                    svf = jnp.broadcast_to(al[head], (16,))
                    sv = plsc.pack(svf, svf,
                                   format=plsc.PackFormat.INTERLEAVED)
                    for blk in range(4):
                        seg = h_v[j, pl.ds(head * D + blk * 32, 32)]
                        if head == 0:
                            acc[blk] = sv * seg
                        else:
                            acc[blk] = acc[blk] + sv * seg
                for blk in range(4):
                    ev, od = plsc.unpack(acc[blk],
                                         format=plsc.PackFormat.INTERLEAVED)
                    ct_v[j, pl.ds(blk * 32, 16)] = ev
                    ct_v[j, pl.ds(blk * 32 + 16, 16)] = od

            pltpu.async_copy(cts[k], acc_sp.at[dscs[k]], sem_ct[k], add=True)

        for k in range(S):
            sd_issue(k, k)
            ex_issue(k, k)
        sd_wait(0)
        g_issue(0)
        sd_wait(1)
        g_issue(1)

        def pipe(i, carry):
            g0 = 3 * i
            for k in range(S):
                g = g0 + k
                kb = (k + 2) % 3
                kp = (k + 2) % 3
                g_wait(k)
                dscs[k][...] = dsts[k][...]
                sd_wait(kb)
                g_issue(kb)
                sd_issue(g + 3, k)
                ex_wait(k)
                compute(k)

                @pl.when(g >= 1)
                def _():
                    ct_drain(kp)

                ex_issue(g + 3, k)
            return carry

        lax.fori_loop(0, (nch - 1) // 3, pipe, 0)
        g_wait(0)
        dscs[0][...] = dsts[0][...]
        ex_wait(0)
        compute(0)
        ct_drain(2)
        ct_drain(0)
        g_wait(1)
        sd_wait(2)
        ex_wait(1)
        ex_wait(2)
        plsc.subcore_barrier()
        _tile_slice_copy(s, lambda o, n: acc_sp.at[pl.ds(o, n)],
                         lambda o, n: out_hbm.at[c].at[pl.ds(o, n)])

    return kern(ei, ex, den, h, z128)


def _layer_sc(h, asp, adp, ei):
    m8 = jnp.maximum(jnp.max(asp[:, :8], axis=0) + jnp.max(adp[:, :8], axis=0),
                     0.0)
    m16 = jnp.concatenate([m8, jnp.zeros((8,), f32)])
    z16 = jnp.zeros((N, 16), f32)
    ex, den = _sc_pass_a(ei, asp, adp, m16, z16)
    z128 = jnp.zeros((N, D), f32)
    return _sc_pass_b(ei, ex, den, h, z128)


def kernel(x, edge_index, W1, a_src1, a_dst1, b1, W2, a_src2, a_dst2, b2):
    permd = jnp.arange(D).reshape(4, 2, 16).transpose(0, 2, 1).reshape(-1)
    permf = (jnp.arange(H)[:, None] * D + permd[None, :]).reshape(-1)

    def pw(W):
        return W[:, permf]

    def pa(a):
        return a[:, permd]

    h1, asp1, adp1 = _dense_entry(x, pw(W1), pa(a_src1), pa(a_dst1))
    p1 = _layer_sc(h1, asp1, adp1, edge_index)
    h2, asp2, adp2 = _dense_combine(p1[0], p1[1], b1.reshape(1, D),
                                    pw(W2), pa(a_src2), pa(a_dst2))
    p2 = _layer_sc(h2, asp2, adp2, edge_index)
    return _final_combine(p2[0], p2[1], b2.reshape(1, D))


# speedup vs baseline: 68.3594x; 1.0018x over previous
import functools

import jax
import jax.numpy as jnp
from jax import lax
from jax.experimental import pallas as pl
from jax.experimental.pallas import tpu as pltpu
from jax.experimental.pallas import tpu_sc as plsc

N = 10000
E = 320000
D = 128
H = 8
HD = H * D

NC = 2
NS = 16
NW = NC * NS
EPW = E // NW
CA = 80
CB = 16
RPT = 640
RPT_LAST = N - (NS - 1) * RPT

f32 = jnp.float32


def _for_tile_slice(s, fn):
    off = pl.multiple_of(s * RPT, 8)
    off_last = (NS - 1) * RPT

    @pl.when(s < NS - 1)
    def _():
        fn(off, RPT)

    @pl.when(s == NS - 1)
    def _():
        fn(off_last, RPT_LAST)


def _tile_slice_copy(s, src_fn, dst_fn):
    _for_tile_slice(
        s, lambda o, n: pltpu.sync_copy(src_fn(o, n), dst_fn(o, n)))


def _dense_body(x, w_ref, as_ref, ad_ref, h_ref, asp_ref, adp_ref, rows):
    h = jnp.dot(x, w_ref[...], preferred_element_type=f32)
    h_ref[...] = h.astype(jnp.bfloat16)
    h3 = h.reshape(rows, H, D)
    ssrc = jnp.sum(h3 * as_ref[...][None], axis=-1)
    sdst = jnp.sum(h3 * ad_ref[...][None], axis=-1)
    z = jnp.zeros_like(ssrc)
    asp_ref[...] = jnp.concatenate([ssrc, z], axis=-1)
    adp_ref[...] = jnp.concatenate([sdst, z], axis=-1)


def _dense_entry(x, W, a_src, a_dst):
    R = 2000

    def body(x_ref, w_ref, as_ref, ad_ref, h_ref, asp_ref, adp_ref):
        _dense_body(x_ref[...], w_ref, as_ref, ad_ref, h_ref, asp_ref,
                    adp_ref, R)

    return pl.pallas_call(
        body,
        grid=(N // R,),
        in_specs=[
            pl.BlockSpec((R, D), lambda i: (i, 0)),
            pl.BlockSpec((D, HD), lambda i: (0, 0)),
            pl.BlockSpec((H, D), lambda i: (0, 0)),
            pl.BlockSpec((H, D), lambda i: (0, 0)),
        ],
        out_specs=[
            pl.BlockSpec((R, HD), lambda i: (i, 0)),
            pl.BlockSpec((R, 16), lambda i: (i, 0)),
            pl.BlockSpec((R, 16), lambda i: (i, 0)),
        ],
        out_shape=[
            jax.ShapeDtypeStruct((N, HD), jnp.bfloat16),
            jax.ShapeDtypeStruct((N, 16), f32),
            jax.ShapeDtypeStruct((N, 16), f32),
        ],
    )(x, W, a_src, a_dst)


def _dense_combine(p0, p1, b2d, W, a_src, a_dst):
    R = 2000

    def body(p0_ref, p1_ref, b_ref, w_ref, as_ref, ad_ref,
             h_ref, asp_ref, adp_ref):
        v = p0_ref[...] + p1_ref[...] + b_ref[...]
        x = jnp.where(v > 0, v, jnp.exp(v) - 1.0)
        _dense_body(x, w_ref, as_ref, ad_ref, h_ref, asp_ref, adp_ref, R)

    return pl.pallas_call(
        body,
        grid=(N // R,),
        in_specs=[
            pl.BlockSpec((R, D), lambda i: (i, 0)),
            pl.BlockSpec((R, D), lambda i: (i, 0)),
            pl.BlockSpec((1, D), lambda i: (0, 0)),
            pl.BlockSpec((D, HD), lambda i: (0, 0)),
            pl.BlockSpec((H, D), lambda i: (0, 0)),
            pl.BlockSpec((H, D), lambda i: (0, 0)),
        ],
        out_specs=[
            pl.BlockSpec((R, HD), lambda i: (i, 0)),
            pl.BlockSpec((R, 16), lambda i: (i, 0)),
            pl.BlockSpec((R, 16), lambda i: (i, 0)),
        ],
        out_shape=[
            jax.ShapeDtypeStruct((N, HD), jnp.bfloat16),
            jax.ShapeDtypeStruct((N, 16), f32),
            jax.ShapeDtypeStruct((N, 16), f32),
        ],
    )(p0, p1, b2d, W, a_src, a_dst)


def _final_combine(p0, p1, b2d):
    R = 2000

    def body(p0_ref, p1_ref, b_ref, y_ref):
        v = p0_ref[...] + p1_ref[...] + b_ref[...]
        y_ref[...] = jnp.where(v > 0, v, jnp.exp(v) - 1.0)

    return pl.pallas_call(
        body,
        grid=(N // R,),
        in_specs=[
            pl.BlockSpec((R, D), lambda i: (i, 0)),
            pl.BlockSpec((R, D), lambda i: (i, 0)),
            pl.BlockSpec((1, D), lambda i: (0, 0)),
        ],
        out_specs=pl.BlockSpec((R, D), lambda i: (i, 0)),
        out_shape=jax.ShapeDtypeStruct((N, D), f32),
    )(p0, p1, b2d)


def _sc_pass_a(ei, asp, adp, m16):
    mesh = plsc.VectorSubcoreMesh(core_axis_name="c", subcore_axis_name="s")

    @functools.partial(
        pl.kernel,
        compiler_params=pltpu.CompilerParams(use_tc_tiling_on_sc=False,
                                             needs_layout_passes=False),
        out_type=(
            jax.ShapeDtypeStruct((E, 16), f32),
            jax.ShapeDtypeStruct((NC, N, 16), f32),
        ),
        mesh=mesh,
        scratch_types=[
            [pltpu.VMEM((CA,), jnp.int32)] * 2,
            [pltpu.VMEM((CA,), jnp.int32)] * 2,
            [pltpu.VMEM((CA, 16), f32)] * 2,
            [pltpu.VMEM((CA, 16), f32)] * 2,
            [pltpu.VMEM((CA, 16), f32)] * 2,
            [pltpu.VMEM((CA,), jnp.int32)] * 2,
            pltpu.VMEM((16,), f32),
            pltpu.VMEM_SHARED((N, 16), f32),
            pltpu.VMEM_SHARED((N, 16), f32),
            pltpu.VMEM_SHARED((N, 16), f32),
            [pltpu.SemaphoreType.DMA] * 2,
            [pltpu.SemaphoreType.DMA] * 2,
            [pltpu.SemaphoreType.DMA] * 2,
            [pltpu.SemaphoreType.DMA] * 2,
            [pltpu.SemaphoreType.DMA] * 2,
            [pltpu.SemaphoreType.DMA] * 2,
        ],
    )
    def kern(ei_hbm, asp_hbm, adp_hbm, m_hbm,
             ex_hbm, den_hbm,
             srcs, dsts, ass, ads, exs, dscs, m_v,
             asp_sp, adp_sp, den_sp,
             sem_src, sem_dst, sem_as, sem_ad, sem_w, sem_sc):
        src_hbm = ei_hbm.at[0]
        dst_hbm = ei_hbm.at[1]
        c = lax.axis_index("c")
        s = lax.axis_index("s")
        wid = s * NC + c
        base = wid * EPW
        nch = EPW // CA
        last = nch - 1
        _tile_slice_copy(s, lambda o, n: asp_hbm.at[pl.ds(o, n)],
                         lambda o, n: asp_sp.at[pl.ds(o, n)])
        _tile_slice_copy(s, lambda o, n: adp_hbm.at[pl.ds(o, n)],
                         lambda o, n: adp_sp.at[pl.ds(o, n)])
        zero16 = jnp.zeros((16,), f32)

        @plsc.parallel_loop(0, CA, unroll=4)
        def _(r):
            exs[0][r, :] = zero16

        def zden(o, n):
            for q in range(n // CA):
                pltpu.async_copy(exs[0],
                                 den_sp.at[pl.ds(o + q * CA, CA)], sem_w[0])
            for q in range(n // CA):
                pltpu.make_async_copy(exs[0], den_sp.at[pl.ds(0, CA)],
                                      sem_w[0]).wait()

        _for_tile_slice(s, zden)
        pltpu.sync_copy(m_hbm, m_v)
        plsc.subcore_barrier()
        mvec = m_v[...]

        def coff(g):
            return pl.multiple_of(base + jnp.minimum(g, last) * CA, 8)

        def sd_issue(g, k):
            off = coff(g)
            pltpu.async_copy(src_hbm.at[pl.ds(off, CA)], srcs[k], sem_src[k])
            pltpu.async_copy(dst_hbm.at[pl.ds(off, CA)], dsts[k], sem_dst[k])

        def sd_wait(k):
            pltpu.make_async_copy(src_hbm.at[pl.ds(0, CA)], srcs[k],
                                  sem_src[k]).wait()
            pltpu.make_async_copy(dst_hbm.at[pl.ds(0, CA)], dsts[k],
                                  sem_dst[k]).wait()

        def g_issue(k):
            pltpu.async_copy(asp_sp.at[srcs[k]], ass[k], sem_as[k])
            pltpu.async_copy(adp_sp.at[dsts[k]], ads[k], sem_ad[k])

        def g_wait(k):
            pltpu.make_async_copy(asp_sp.at[srcs[k]], ass[k],
                                  sem_as[k]).wait()
            pltpu.make_async_copy(adp_sp.at[dsts[k]], ads[k],
                                  sem_ad[k]).wait()

        def out_drain(k):
            pltpu.make_async_copy(exs[k], ex_hbm.at[pl.ds(0, CA)],
                                  sem_w[k]).wait()
            pltpu.make_async_copy(exs[k], den_sp.at[dscs[k]],
                                  sem_sc[k]).wait()

        def compute(g, k):
            as_v, ad_v, ex_v = ass[k], ads[k], exs[k]

            @plsc.parallel_loop(0, CA, unroll=2)
            def _(j):
                e = as_v[j, :] + ad_v[j, :]
                e = jnp.maximum(e, 0.2 * e) - mvec
                ex_v[j, :] = jnp.exp(e)

            pltpu.async_copy(ex_v, ex_hbm.at[pl.ds(coff(g), CA)], sem_w[k])
            pltpu.async_copy(ex_v, den_sp.at[dscs[k]], sem_sc[k], add=True)

        sd_issue(0, 0)
        sd_issue(1, 1)
        sd_wait(0)
        g_issue(0)

        def pipe(i, carry):
            g0 = 2 * i
            for k in (0, 1):
                g = g0 + k
                nk = 1 - k
                g_wait(k)
                for q in range(CA // 16):
                    dscs[k][pl.ds(q * 16, 16)] = dsts[k][pl.ds(q * 16, 16)]
                sd_wait(nk)
                g_issue(nk)
                sd_issue(g + 2, k)
                compute(g, k)

                @pl.when(g >= 1)
                def _():
                    out_drain(nk)
            return carry

        lax.fori_loop(0, (nch - 1) // 2, pipe, 0)
        g_wait(0)
        for q in range(CA // 16):
            dscs[0][pl.ds(q * 16, 16)] = dsts[0][pl.ds(q * 16, 16)]
        compute(nch - 1, 0)
        sd_wait(1)
        out_drain(1)
        out_drain(0)
        plsc.subcore_barrier()
        _tile_slice_copy(s, lambda o, n: den_sp.at[pl.ds(o, n)],
                         lambda o, n: den_hbm.at[c].at[pl.ds(o, n)])

    return kern(ei, asp, adp, m16)


def _sc_pass_b(ei, ex, den, h):
    mesh = plsc.VectorSubcoreMesh(core_axis_name="c", subcore_axis_name="s")
    S = 3

    @functools.partial(
        pl.kernel,
        compiler_params=pltpu.CompilerParams(use_tc_tiling_on_sc=False,
                                             needs_layout_passes=False),
        out_type=jax.ShapeDtypeStruct((NC, N, D), f32),
        mesh=mesh,
        scratch_types=[
            [pltpu.VMEM((CB,), jnp.int32)] * S,
            [pltpu.VMEM((CB,), jnp.int32)] * S,
            [pltpu.VMEM((CB, 16), f32)] * S,
            [pltpu.VMEM((CB, 16), f32)] * S,
            [pltpu.VMEM((CB, HD), jnp.bfloat16)] * S,
            [pltpu.VMEM((CB, D), f32)] * S,
            [pltpu.VMEM((CB,), jnp.int32)] * S,
            pltpu.VMEM((16, 16), f32),
            pltpu.VMEM((16, 16), f32),
            pltpu.VMEM_SHARED((N, 16), f32),
            pltpu.VMEM_SHARED((N, D), f32),
            [pltpu.SemaphoreType.DMA] * S,
            [pltpu.SemaphoreType.DMA] * S,
            [pltpu.SemaphoreType.DMA] * S,
            [pltpu.SemaphoreType.DMA] * S,
            [[pltpu.SemaphoreType.DMA] * S] * 2,
            [pltpu.SemaphoreType.DMA] * S,
        ],
    )
    def kern(ei_hbm, ex_hbm, den_hbm, h_hbm,
             out_hbm,
             srcs, dsts, exs, dns, hs, cts, dscs, t0_v, t1_v,
             den_sp, acc_sp, sem_src, sem_dst, sem_ex, sem_dn, sem_h,
             sem_ct):
        src_hbm = ei_hbm.at[0]
        dst_hbm = ei_hbm.at[1]
        c = lax.axis_index("c")
        s = lax.axis_index("s")
        wid = s * NC + c
        base = wid * EPW
        nch = EPW // CB
        last = nch - 1

        def den_combine(o, n):
            def sub(k, carry):
                oo = pl.multiple_of(o + k * 16, 8)
                pltpu.sync_copy(den_hbm.at[0].at[pl.ds(oo, 16)], t0_v)
                pltpu.sync_copy(den_hbm.at[1].at[pl.ds(oo, 16)], t1_v)

                def add_row(r, carry2):
                    t0_v[r, :] = t0_v[r, :] + t1_v[r, :]
                    return carry2

                lax.fori_loop(0, 16, add_row, 0)
                pltpu.sync_copy(t0_v, den_sp.at[pl.ds(oo, 16)])
                return carry

            lax.fori_loop(0, n // 16, sub, 0)

        _for_tile_slice(s, den_combine)
        zero16 = jnp.zeros((16,), f32)

        @plsc.parallel_loop(0, CB, unroll=2)
        def _(r):
            for q in range(8):
                cts[0][r, pl.ds(q * 16, 16)] = zero16

        def zacc(o, n):
            for q in range(n // CB):
                pltpu.async_copy(cts[0],
                                 acc_sp.at[pl.ds(o + q * CB, CB)], sem_ct[0])
            for q in range(n // CB):
                pltpu.make_async_copy(cts[0], acc_sp.at[pl.ds(0, CB)],
                                      sem_ct[0]).wait()

        _for_tile_slice(s, zacc)
        plsc.subcore_barrier()

        HB = CB // 2

        def coff(g):
            return pl.multiple_of(base + jnp.minimum(g, last) * CB, 8)

        def sd_issue(g, k):
            off = coff(g)
            pltpu.async_copy(src_hbm.at[pl.ds(off, CB)], srcs[k], sem_src[k])
            pltpu.async_copy(dst_hbm.at[pl.ds(off, CB)], dsts[k], sem_dst[k])

        def sd_wait(k):
            pltpu.make_async_copy(src_hbm.at[pl.ds(0, CB)], srcs[k],
                                  sem_src[k]).wait()
            pltpu.make_async_copy(dst_hbm.at[pl.ds(0, CB)], dsts[k],
                                  sem_dst[k]).wait()

        def ex_issue(g, k):
            pltpu.async_copy(ex_hbm.at[pl.ds(coff(g), CB)], exs[k], sem_ex[k])

        def ex_wait(k):
            pltpu.make_async_copy(ex_hbm.at[pl.ds(0, CB)], exs[k],
                                  sem_ex[k]).wait()

        def g_issue(k):
            pltpu.async_copy(den_sp.at[dsts[k]], dns[k], sem_dn[k])
            for q in range(2):
                pltpu.async_copy(h_hbm.at[srcs[k].at[pl.ds(q * HB, HB)]],
                                 hs[k].at[pl.ds(q * HB, HB)], sem_h[q][k])

        def g_wait(k):
            pltpu.make_async_copy(den_sp.at[dsts[k]], dns[k], sem_dn[k]).wait()
            for q in range(2):
                pltpu.make_async_copy(h_hbm.at[srcs[k].at[pl.ds(q * HB, HB)]],
                                      hs[k].at[pl.ds(q * HB, HB)],
                                      sem_h[q][k]).wait()

        def ct_drain(k):
            pltpu.make_async_copy(cts[k], acc_sp.at[dscs[k]],
                                  sem_ct[k]).wait()

        def compute(k):
            ex_v, dn_v, h_v, ct_v = exs[k], dns[k], hs[k], cts[k]

            @plsc.parallel_loop(0, CB, unroll=2)
            def _(j):
                al = ex_v[j, :] / (dn_v[j, :] + 1e-16)
                al = al * (1.0 / H)
                acc = [None] * 4
                for head in range(H):
                    svf = jnp.broadcast_to(al[head], (16,))
                    sv = plsc.pack(svf, svf,
                                   format=plsc.PackFormat.INTERLEAVED)
                    for blk in range(4):
                        seg = h_v[j, pl.ds(head * D + blk * 32, 32)]
                        if head == 0:
                            acc[blk] = sv * seg
                        else:
                            acc[blk] = acc[blk] + sv * seg
                for blk in range(4):
                    ev, od = plsc.unpack(acc[blk],
                                         format=plsc.PackFormat.INTERLEAVED)
                    ct_v[j, pl.ds(blk * 32, 16)] = ev
                    ct_v[j, pl.ds(blk * 32 + 16, 16)] = od

            pltpu.async_copy(cts[k], acc_sp.at[dscs[k]], sem_ct[k], add=True)

        for k in range(S):
            sd_issue(k, k)
            ex_issue(k, k)
        sd_wait(0)
        g_issue(0)
        sd_wait(1)
        g_issue(1)

        def pipe(i, carry):
            g0 = 3 * i
            for k in range(S):
                g = g0 + k
                kb = (k + 2) % 3
                kp = (k + 2) % 3
                g_wait(k)
                dscs[k][...] = dsts[k][...]
                sd_wait(kb)
                g_issue(kb)
                sd_issue(g + 3, k)
                ex_wait(k)
                compute(k)

                @pl.when(g >= 1)
                def _():
                    ct_drain(kp)

                ex_issue(g + 3, k)
            return carry

        lax.fori_loop(0, (nch - 1) // 3, pipe, 0)
        g_wait(0)
        dscs[0][...] = dsts[0][...]
        ex_wait(0)
        compute(0)
        ct_drain(2)
        ct_drain(0)
        g_wait(1)
        sd_wait(2)
        ex_wait(1)
        ex_wait(2)
        plsc.subcore_barrier()
        _tile_slice_copy(s, lambda o, n: acc_sp.at[pl.ds(o, n)],
                         lambda o, n: out_hbm.at[c].at[pl.ds(o, n)])

    return kern(ei, ex, den, h)


def _layer_sc(h, asp, adp, ei):
    m8 = jnp.maximum(jnp.max(asp[:, :8], axis=0) + jnp.max(adp[:, :8], axis=0),
                     0.0)
    m16 = jnp.concatenate([m8, jnp.zeros((8,), f32)])
    ex, den = _sc_pass_a(ei, asp, adp, m16)
    return _sc_pass_b(ei, ex, den, h)


def kernel(x, edge_index, W1, a_src1, a_dst1, b1, W2, a_src2, a_dst2, b2):
    permd = jnp.arange(D).reshape(4, 2, 16).transpose(0, 2, 1).reshape(-1)
    permf = (jnp.arange(H)[:, None] * D + permd[None, :]).reshape(-1)

    def pw(W):
        return W[:, permf]

    def pa(a):
        return a[:, permd]

    h1, asp1, adp1 = _dense_entry(x, pw(W1), pa(a_src1), pa(a_dst1))
    p1 = _layer_sc(h1, asp1, adp1, edge_index)
    h2, asp2, adp2 = _dense_combine(p1[0], p1[1], b1.reshape(1, D),
                                    pw(W2), pa(a_src2), pa(a_dst2))
    p2 = _layer_sc(h2, asp2, adp2, edge_index)
    return _final_combine(p2[0], p2[1], b2.reshape(1, D))
